# baseline (speedup 1.0000x reference)
import jax
import jax.numpy as jnp
from jax.experimental import pallas as pl
from jax.experimental.pallas import tpu as pltpu


def _mm_kernel(x_ref, y_ref, o_ref):
    o_ref[0] = jnp.dot(
        x_ref[0],
        y_ref[0],
        preferred_element_type=jnp.float32,
    )


def kernel(Y, x):
    B, N, _ = Y.shape
    _, C, F, _ = x.shape
    M = C * F
    x2 = x.reshape(B, M, N).astype(jnp.bfloat16)
    TN = 2048
    out = pl.pallas_call(
        _mm_kernel,
        grid=(B, N // TN),
        in_specs=[
            pl.BlockSpec((1, M, N), lambda b, j: (b, 0, 0)),
            pl.BlockSpec((1, N, TN), lambda b, j: (b, 0, j)),
        ],
        out_specs=pl.BlockSpec((1, M, TN), lambda b, j: (b, 0, j)),
        out_shape=jax.ShapeDtypeStruct((B, M, N), jnp.float32),
        compiler_params=pltpu.CompilerParams(
            allow_input_fusion=[False, True],
        ),
    )(x2, Y.astype(jnp.bfloat16))
    return out.reshape(B, C, F, N)


# speedup vs baseline: 1.8235x; 1.8235x over previous
import jax
import jax.numpy as jnp
from jax.experimental import pallas as pl
from jax.experimental.pallas import tpu as pltpu


def _mm_kernel(x_ref, y0_ref, y1_ref, y2_ref, y3_ref, o_ref):
    for b, y_ref in enumerate((y0_ref, y1_ref, y2_ref, y3_ref)):
        o_ref[b] = jnp.dot(
            x_ref[b],
            y_ref[0],
            preferred_element_type=jnp.float32,
        )


def kernel(Y, x):
    B, N, _ = Y.shape
    _, C, F, _ = x.shape
    M = C * F
    x2 = x.reshape(B, M, N).astype(jnp.bfloat16)
    Yb = Y.astype(jnp.bfloat16)
    TN = 512

    def y_spec(b):
        return pl.BlockSpec((1, N, TN), lambda j, b=b: (b, 0, j))

    out = pl.pallas_call(
        _mm_kernel,
        grid=(N // TN,),
        in_specs=[pl.BlockSpec((B, M, N), lambda j: (0, 0, 0))]
        + [y_spec(b) for b in range(B)],
        out_specs=pl.BlockSpec((B, M, TN), lambda j: (0, 0, j)),
        out_shape=jax.ShapeDtypeStruct((B, M, N), jnp.float32),
        compiler_params=pltpu.CompilerParams(
            allow_input_fusion=[False, True, True, True, True],
        ),
    )(x2, Yb, Yb, Yb, Yb)
    return out.reshape(B, C, F, N)
